# baseline (speedup 1.0000x reference)
import functools

import jax
import jax.numpy as jnp
from jax import lax
from jax.experimental import pallas as pl
from jax.experimental.pallas import tpu as pltpu
from jax.experimental.pallas import tpu_sc as plsc

_CHANS = 64
_SAMPLES = 1024
_DIM = 128
_BATCH = 4
_NC = 2
_NS = 16
_NW = _NC * _NS
_SPW = (_BATCH * _SAMPLES) // _NW
_SCB = 2
_NSLAB = _SPW // _SCB
_GS = 2
_NGRP = _NSLAB // _GS


def _sc_body(xt_hbm, t_hbm, cb_hbm, w_hbm, out_hbm,
             x_v, t_v, cb_v, w_v, out_v, x_sems, t_sems, out_sems):
    wid = lax.axis_index("s") * _NC + lax.axis_index("c")
    b_idx = wid // (_SAMPLES // _SPW)
    s0 = (wid % (_SAMPLES // _SPW)) * _SPW

    pltpu.sync_copy(cb_hbm, cb_v)
    pltpu.sync_copy(w_hbm, w_v)

    w_regs = [w_v[pl.ds(j * 16, 16)] for j in range(_DIM // 16)]

    def compute(gb, sl_off, buf):
        @plsc.parallel_loop(0, _SCB)
        def sample_body(sl):
            tb = [t_v[gb, sl_off + sl, pl.ds(j * 16, 16)] for j in range(_DIM // 16)]

            @plsc.parallel_loop(0, _CHANS // 16, unroll=2)
            def chan_body(c16):
                tok16 = (sl_off + sl) * _CHANS + c16 * 16
                xrow = x_v[gb, pl.ds(tok16, 16)]
                for ci in range(16):
                    xv = jnp.full((16,), xrow[ci], dtype=jnp.float32)
                    c = c16 * 16 + ci
                    tok = tok16 + ci - sl_off * _CHANS
                    vals = [xv * w_regs[j] + (tb[j] + cb_v[c, pl.ds(j * 16, 16)])
                            for j in range(_DIM // 16)]
                    for j in range(_DIM // 16):
                        out_v[buf, tok, pl.ds(j * 16, 16)] = vals[j]

    def start_group_in(gi, gb):
        pltpu.async_copy(
            xt_hbm.at[b_idx,
                      pl.ds(s0 * _CHANS + gi * (_GS * _SCB * _CHANS),
                            _GS * _SCB * _CHANS)],
            x_v.at[gb], x_sems.at[gb])
        pltpu.async_copy(
            t_hbm.at[pl.ds(s0 + gi * (_GS * _SCB), _GS * _SCB), :],
            t_v.at[gb], t_sems.at[gb])

    start_group_in(0, 0)

    def group_body(gi, carry):
        gb = lax.rem(gi, 2)
        grp0 = s0 * _CHANS + gi * (_GS * _SCB * _CHANS)

        @pl.when(gi + 1 < _NGRP)
        def _():
            start_group_in(gi + 1, 1 - gb)

        pltpu.make_async_copy(
            xt_hbm.at[b_idx, pl.ds(grp0, _GS * _SCB * _CHANS)],
            x_v.at[gb], x_sems.at[gb]).wait()
        pltpu.make_async_copy(
            t_hbm.at[pl.ds(s0 + gi * (_GS * _SCB), _GS * _SCB), :],
            t_v.at[gb], t_sems.at[gb]).wait()
        for sb in range(_GS):
            buf = sb
            tok0 = grp0 + sb * (_SCB * _CHANS)
            dst = out_hbm.at[b_idx, pl.ds(tok0, _SCB * _CHANS), :]

            @pl.when(gi > 0)
            def _():
                pltpu.make_async_copy(
                    out_v.at[buf], dst, out_sems.at[buf]).wait()

            compute(gb, sb * _SCB, buf)
            pltpu.async_copy(out_v.at[buf], dst, out_sems.at[buf])
        return carry

    lax.fori_loop(0, _NGRP, group_body, 0)
    for buf in range(_GS):
        tok0 = s0 * _CHANS + (_NSLAB - _GS + buf) * (_SCB * _CHANS)
        pltpu.make_async_copy(
            out_v.at[buf],
            out_hbm.at[b_idx, pl.ds(tok0, _SCB * _CHANS), :],
            out_sems.at[buf]).wait()


@functools.partial(jax.jit, static_argnames=())
def kernel(x, t_table, c_table, W, b):
    xt = jnp.transpose(x[:, 0], (0, 2, 1)).reshape(_BATCH, _SAMPLES * _CHANS)
    cb = c_table + b[None, :]
    wv = W[:, 0]
    mesh = plsc.VectorSubcoreMesh(
        core_axis_name="c", subcore_axis_name="s",
        num_cores=_NC, num_subcores=_NS)
    f = pl.kernel(
        _sc_body,
        out_type=jax.ShapeDtypeStruct((_BATCH, _SAMPLES * _CHANS, _DIM), jnp.float32),
        mesh=mesh,
        scratch_types=[
            pltpu.VMEM((2, _GS * _SCB * _CHANS), jnp.float32),
            pltpu.VMEM((2, _GS * _SCB, _DIM), jnp.float32),
            pltpu.VMEM((_CHANS, _DIM), jnp.float32),
            pltpu.VMEM((_DIM,), jnp.float32),
            pltpu.VMEM((_GS, _SCB * _CHANS, _DIM), jnp.float32),
            pltpu.SemaphoreType.DMA((2,)),
            pltpu.SemaphoreType.DMA((2,)),
            pltpu.SemaphoreType.DMA((_GS,)),
        ],
    )
    return f(xt, t_table, cb, wv)


# speedup vs baseline: 1.1721x; 1.1721x over previous
import functools

import jax
import jax.numpy as jnp
from jax import lax
from jax.experimental import pallas as pl
from jax.experimental.pallas import tpu as pltpu
from jax.experimental.pallas import tpu_sc as plsc

_CHANS = 64
_SAMPLES = 1024
_DIM = 128
_BATCH = 4
_NC = 2
_NS = 16
_NW = _NC * _NS
_SPW = (_BATCH * _SAMPLES) // _NW
_SCB = 1
_NSLAB = _SPW // _SCB
_GS = 4
_NGRP = _NSLAB // _GS


def _sc_body(xt_hbm, t_hbm, cb_hbm, w_hbm, out_hbm,
             x_v, t_v, cb_v, w_v, out_v, x_sems, t_sems, out_sems):
    wid = lax.axis_index("s") * _NC + lax.axis_index("c")
    b_idx = wid // (_SAMPLES // _SPW)
    s0 = (wid % (_SAMPLES // _SPW)) * _SPW

    pltpu.sync_copy(cb_hbm, cb_v)
    pltpu.sync_copy(w_hbm, w_v)

    w_regs = [w_v[pl.ds(j * 16, 16)] for j in range(_DIM // 16)]

    def compute(gb, sl_off, buf):
        @plsc.parallel_loop(0, _SCB)
        def sample_body(sl):
            tb = [t_v[gb, sl_off + sl, pl.ds(j * 16, 16)] for j in range(_DIM // 16)]

            @plsc.parallel_loop(0, _CHANS // 16, unroll=2)
            def chan_body(c16):
                tok16 = (sl_off + sl) * _CHANS + c16 * 16
                xrow = x_v[gb, pl.ds(tok16, 16)]
                for ci in range(16):
                    xv = jnp.full((16,), xrow[ci], dtype=jnp.float32)
                    c = c16 * 16 + ci
                    tok = tok16 + ci - sl_off * _CHANS
                    vals = [xv * w_regs[j] + (tb[j] + cb_v[c, pl.ds(j * 16, 16)])
                            for j in range(_DIM // 16)]
                    for j in range(_DIM // 16):
                        out_v[buf, tok, pl.ds(j * 16, 16)] = vals[j]

    def start_group_in(gi, gb):
        pltpu.async_copy(
            xt_hbm.at[b_idx,
                      pl.ds(s0 * _CHANS + gi * (_GS * _SCB * _CHANS),
                            _GS * _SCB * _CHANS)],
            x_v.at[gb], x_sems.at[gb])
        pltpu.async_copy(
            t_hbm.at[pl.ds(s0 + gi * (_GS * _SCB), _GS * _SCB), :],
            t_v.at[gb], t_sems.at[gb])

    start_group_in(0, 0)

    def group_body(gi, carry):
        gb = lax.rem(gi, 2)
        grp0 = s0 * _CHANS + gi * (_GS * _SCB * _CHANS)

        @pl.when(gi + 1 < _NGRP)
        def _():
            start_group_in(gi + 1, 1 - gb)

        pltpu.make_async_copy(
            xt_hbm.at[b_idx, pl.ds(grp0, _GS * _SCB * _CHANS)],
            x_v.at[gb], x_sems.at[gb]).wait()
        pltpu.make_async_copy(
            t_hbm.at[pl.ds(s0 + gi * (_GS * _SCB), _GS * _SCB), :],
            t_v.at[gb], t_sems.at[gb]).wait()
        for sb in range(_GS):
            buf = sb
            tok0 = grp0 + sb * (_SCB * _CHANS)
            dst = out_hbm.at[b_idx, pl.ds(tok0, _SCB * _CHANS), :]

            @pl.when(gi > 0)
            def _():
                pltpu.make_async_copy(
                    out_v.at[buf], dst, out_sems.at[buf]).wait()

            compute(gb, sb * _SCB, buf)
            pltpu.async_copy(out_v.at[buf], dst, out_sems.at[buf])
        return carry

    lax.fori_loop(0, _NGRP, group_body, 0)
    for buf in range(_GS):
        tok0 = s0 * _CHANS + (_NSLAB - _GS + buf) * (_SCB * _CHANS)
        pltpu.make_async_copy(
            out_v.at[buf],
            out_hbm.at[b_idx, pl.ds(tok0, _SCB * _CHANS), :],
            out_sems.at[buf]).wait()


@functools.partial(jax.jit, static_argnames=())
def kernel(x, t_table, c_table, W, b):
    xt = jnp.transpose(x[:, 0], (0, 2, 1)).reshape(_BATCH, _SAMPLES * _CHANS)
    cb = c_table + b[None, :]
    wv = W[:, 0]
    mesh = plsc.VectorSubcoreMesh(
        core_axis_name="c", subcore_axis_name="s",
        num_cores=_NC, num_subcores=_NS)
    f = pl.kernel(
        _sc_body,
        out_type=jax.ShapeDtypeStruct((_BATCH, _SAMPLES * _CHANS, _DIM), jnp.float32),
        mesh=mesh,
        scratch_types=[
            pltpu.VMEM((2, _GS * _SCB * _CHANS), jnp.float32),
            pltpu.VMEM((2, _GS * _SCB, _DIM), jnp.float32),
            pltpu.VMEM((_CHANS, _DIM), jnp.float32),
            pltpu.VMEM((_DIM,), jnp.float32),
            pltpu.VMEM((_GS, _SCB * _CHANS, _DIM), jnp.float32),
            pltpu.SemaphoreType.DMA((2,)),
            pltpu.SemaphoreType.DMA((2,)),
            pltpu.SemaphoreType.DMA((_GS,)),
        ],
    )
    return f(xt, t_table, cb, wv)
